# Initial kernel scaffold; baseline (speedup 1.0000x reference)
import jax
import jax.numpy as jnp
from jax.experimental import pallas as pl


def kernel(support_xyz, support_features, batch_num_xyzs, new_xyz, batch_num_new_xyzs, W1, g1, b1, W2, g2, b2):
    raise NotImplementedError("write your pallas kernel here")


# speedup vs baseline: 2.3431x; 2.3431x over previous
import jax
import jax.numpy as jnp
import numpy as np
from jax.experimental import pallas as pl
from jax.experimental.pallas import tpu as pltpu

N = 16384
M = 4096
B = 2
C_IN = 32
NUM_REDUCED = 2
TOTAL_VOX = 27
R = 1.2
MULT = 2.0
C_AGG = 32
POST = 128
CR = NUM_REDUCED + 9

NB = N // B
CEN = M * TOTAL_VOX
CPB = CEN // B
RAD2 = (R * MULT) ** 2
BIG = 1e30
TAKEN = 2e30

CBLK = 64
SK = 512
CATW = 3 + C_IN


def _voxel_offs():
    g = jnp.array([-2.0 * R / 3.0, 0.0, 2.0 * R / 3.0], dtype=jnp.float32)
    xx, yy, zz = jnp.meshgrid(g, g, g, indexing='ij')
    return jnp.stack([xx, yy, zz], axis=-1).reshape(-1, 3)


def _nn_body(cen_ref, supT_ref, cat_ref, out_ref):
    nchunk = NB // SK
    cen = cen_ref[...]
    cx = cen[:, 0:1]
    cy = cen[:, 1:2]
    cz = cen[:, 2:3]
    inf = jnp.full((CBLK, 1), jnp.inf, jnp.float32)
    zi = jnp.zeros((CBLK, 1), jnp.int32)
    d1, d2v, d3 = inf, inf, inf
    a1, a2v, a3 = zi, zi, zi
    lane = jax.lax.broadcasted_iota(jnp.int32, (CBLK, SK), 1)
    for ci in range(nchunk):
        sx = supT_ref[0, 0:1, pl.ds(ci * SK, SK)]
        sy = supT_ref[0, 1:2, pl.ds(ci * SK, SK)]
        sz = supT_ref[0, 2:3, pl.ds(ci * SK, SK)]
        dx = cx - sx
        dy = cy - sy
        dz = cz - sz
        dd = dx * dx + dy * dy + dz * dz
        dd = jnp.where(dd <= jnp.float32(RAD2), dd, jnp.float32(BIG))
        idx = lane + ci * SK
        for k in range(3):
            m = jnp.min(dd, axis=1, keepdims=True)
            am = jnp.min(jnp.where(dd == m, idx, N), axis=1, keepdims=True)
            if k < 2:
                dd = jnp.where(idx == am, jnp.float32(TAKEN), dd)
            lt1 = m < d1
            lt2 = m < d2v
            lt3 = m < d3
            nd1 = jnp.where(lt1, m, d1)
            na1 = jnp.where(lt1, am, a1)
            nd2 = jnp.where(lt1, d1, jnp.where(lt2, m, d2v))
            na2 = jnp.where(lt1, a1, jnp.where(lt2, am, a2v))
            nd3 = jnp.where(lt2, d2v, jnp.where(lt3, m, d3))
            na3 = jnp.where(lt2, a2v, jnp.where(lt3, am, a3))
            d1, d2v, d3, a1, a2v, a3 = nd1, nd2, nd3, na1, na2, na3
    g1 = jnp.zeros((CBLK, CATW), jnp.float32)
    g2 = jnp.zeros((CBLK, CATW), jnp.float32)
    g3 = jnp.zeros((CBLK, CATW), jnp.float32)
    for ci in range(nchunk):
        idx = lane + ci * SK
        catc = cat_ref[0, pl.ds(ci * SK, SK), :]
        g1 = g1 + jnp.dot((idx == a1).astype(jnp.float32), catc,
                          preferred_element_type=jnp.float32,
                          precision=jax.lax.Precision.HIGHEST)
        g2 = g2 + jnp.dot((idx == a2v).astype(jnp.float32), catc,
                          preferred_element_type=jnp.float32,
                          precision=jax.lax.Precision.HIGHEST)
        g3 = g3 + jnp.dot((idx == a3).astype(jnp.float32), catc,
                          preferred_element_type=jnp.float32,
                          precision=jax.lax.Precision.HIGHEST)
    s1 = jnp.minimum(jnp.sqrt(jnp.maximum(d1, 0.0)), jnp.float32(1e8))
    s2 = jnp.minimum(jnp.sqrt(jnp.maximum(d2v, 0.0)), jnp.float32(1e8))
    s3 = jnp.minimum(jnp.sqrt(jnp.maximum(d3, 0.0)), jnp.float32(1e8))
    r1 = 1.0 / (s1 + jnp.float32(1e-8))
    r2 = 1.0 / (s2 + jnp.float32(1e-8))
    r3 = 1.0 / (s3 + jnp.float32(1e-8))
    norm = jnp.maximum(r1 + r2 + r3, jnp.float32(1e-8))
    w1 = r1 / norm
    w2 = r2 / norm
    w3 = r3 / norm
    def _fr(g):
        acc = g[:, 3:5]
        for j in range(1, C_IN // NUM_REDUCED):
            acc = acc + g[:, 3 + 2 * j:5 + 2 * j]
        return acc
    interp = w1 * _fr(g1) + w2 * _fr(g2) + w3 * _fr(g3)
    loc1 = cen - g1[:, 0:3]
    loc2 = cen - g2[:, 0:3]
    loc3 = cen - g3[:, 0:3]
    nf = jnp.concatenate([interp, loc1, loc2, loc3], axis=1)
    empty = d1 > jnp.float32(1e20)
    out_ref[...] = jnp.where(empty, 0.0, nf)


def _run_nn(cflat, supT, cat, interpret=False):
    nblk = CEN // CBLK
    bpb = CPB // CBLK
    return pl.pallas_call(
        _nn_body,
        grid=(nblk,),
        in_specs=[
            pl.BlockSpec((CBLK, 3), lambda i: (i, 0)),
            pl.BlockSpec((1, 3, NB), lambda i: (i // bpb, 0, 0)),
            pl.BlockSpec((1, NB, CATW), lambda i: (i // bpb, 0, 0)),
        ],
        out_specs=pl.BlockSpec((CBLK, 11), lambda i: (i, 0)),
        out_shape=jax.ShapeDtypeStruct((CEN, 11), jnp.float32),
        interpret=interpret,
    )(cflat, supT, cat)


def _mlp_a_body(nf_ref, W1_ref, g1_ref, b1_ref, W2_ref, out_ref):
    v = pl.program_id(0)
    RB = min(512, M)
    w1 = W1_ref[0]
    g1v = g1_ref[0]
    b1v = b1_ref[0]
    s = jnp.zeros((1, C_AGG), jnp.float32)
    s2 = jnp.zeros((1, C_AGG), jnp.float32)
    hs = []
    for rb in range(M // RB):
        x = nf_ref[0, pl.ds(rb * RB, RB), :]
        h = jnp.dot(x, w1, preferred_element_type=jnp.float32)
        hs.append(h)
        s = s + jnp.sum(h, axis=0, keepdims=True)
        s2 = s2 + jnp.sum(h * h, axis=0, keepdims=True)
    mu = s / M
    var = s2 / M - mu * mu
    scale = g1v / jnp.sqrt(var + jnp.float32(1e-5))
    shift = b1v - mu * scale
    w2 = W2_ref[...]
    for rb in range(M // RB):
        hr = jnp.maximum(hs[rb] * scale + shift, 0.0)
        part = jnp.dot(hr, w2, preferred_element_type=jnp.float32)
        prev = out_ref[pl.ds(rb * RB, RB), :]
        out_ref[pl.ds(rb * RB, RB), :] = jnp.where(v == 0, part, prev + part)


def _mlp_b_body(acc_ref, g2_ref, b2_ref, out_ref):
    RB = min(512, M)
    s = jnp.zeros((1, POST), jnp.float32)
    s2 = jnp.zeros((1, POST), jnp.float32)
    for rb in range(M // RB):
        h = acc_ref[pl.ds(rb * RB, RB), :]
        s = s + jnp.sum(h, axis=0, keepdims=True)
        s2 = s2 + jnp.sum(h * h, axis=0, keepdims=True)
    mu = s / M
    var = s2 / M - mu * mu
    scale = g2_ref[...] / jnp.sqrt(var + jnp.float32(1e-5))
    shift = b2_ref[...] - mu * scale
    for rb in range(M // RB):
        h = acc_ref[pl.ds(rb * RB, RB), :]
        out_ref[pl.ds(rb * RB, RB), :] = jnp.maximum(h * scale + shift, 0.0)


def _run_mlp(nfT, W1, g1, b1, W2, g2, b2, interpret=False):
    acc = pl.pallas_call(
        _mlp_a_body,
        grid=(TOTAL_VOX,),
        in_specs=[
            pl.BlockSpec((1, M, CR), lambda v: (v, 0, 0)),
            pl.BlockSpec((1, CR, C_AGG), lambda v: (v, 0, 0)),
            pl.BlockSpec((1, 1, C_AGG), lambda v: (v, 0, 0)),
            pl.BlockSpec((1, 1, C_AGG), lambda v: (v, 0, 0)),
            pl.BlockSpec((C_AGG, POST), lambda v: (v, 0)),
        ],
        out_specs=pl.BlockSpec((M, POST), lambda v: (0, 0)),
        out_shape=jax.ShapeDtypeStruct((M, POST), jnp.float32),
        interpret=interpret,
    )(nfT, W1, g1.reshape(TOTAL_VOX, 1, C_AGG), b1.reshape(TOTAL_VOX, 1, C_AGG),
      W2)
    return pl.pallas_call(
        _mlp_b_body,
        in_specs=[
            pl.BlockSpec((M, POST), lambda: (0, 0)),
            pl.BlockSpec((1, POST), lambda: (0, 0)),
            pl.BlockSpec((1, POST), lambda: (0, 0)),
        ],
        out_specs=pl.BlockSpec((M, POST), lambda: (0, 0)),
        out_shape=jax.ShapeDtypeStruct((M, POST), jnp.float32),
        interpret=interpret,
    )(acc, g2.reshape(1, POST), b2.reshape(1, POST))


def _impl(support_xyz, support_features, new_xyz, W1, g1, b1, W2, g2, b2,
          interpret=False):
    centers = new_xyz[:, None, :] + _voxel_offs()[None, :, :]
    cflat = centers.reshape(-1, 3)
    supT = support_xyz.reshape(B, NB, 3).transpose(0, 2, 1)
    cat = jnp.concatenate([support_xyz, support_features],
                          axis=1).reshape(B, NB, CATW)
    nf = _run_nn(cflat, supT, cat, interpret=interpret)
    nfT = nf.reshape(M, TOTAL_VOX, CR).transpose(1, 0, 2)
    return _run_mlp(nfT, W1, g1, b1, W2, g2, b2, interpret=interpret)


def kernel(support_xyz, support_features, batch_num_xyzs, new_xyz,
           batch_num_new_xyzs, W1, g1, b1, W2, g2, b2):
    return _impl(support_xyz, support_features, new_xyz, W1, g1, b1, W2, g2,
                 b2)


# speedup vs baseline: 4.2241x; 1.8028x over previous
import functools

import jax
import jax.numpy as jnp
from jax import lax
from jax.experimental import pallas as pl
from jax.experimental.pallas import tpu as pltpu
from jax.experimental.pallas import tpu_sc as plsc

N = 16384
M = 4096
B = 2
C_IN = 32
NUM_REDUCED = 2
TOTAL_VOX = 27
R = 1.2
MULT = 2.0
C_AGG = 32
POST = 128
CR = NUM_REDUCED + 9

NB = N // B
CEN = M * TOTAL_VOX
CPB = CEN // B
RAD2 = (R * MULT) ** 2
BIG = 1e30
TAKEN = 2e30

CBLK = 64
SK = 512

NWORK = 32
SC_CHUNK = 1152


def _voxel_offs():
    g = jnp.array([-2.0 * R / 3.0, 0.0, 2.0 * R / 3.0], dtype=jnp.float32)
    xx, yy, zz = jnp.meshgrid(g, g, g, indexing='ij')
    return jnp.stack([xx, yy, zz], axis=-1).reshape(-1, 3)


def _table_body(xyzT_ref, featT_ref, out_ref):
    out_ref[0:3, :] = xyzT_ref[...]
    f0 = featT_ref[0:1, :]
    f1 = featT_ref[1:2, :]
    for j in range(1, C_IN // NUM_REDUCED):
        f0 = f0 + featT_ref[2 * j:2 * j + 1, :]
        f1 = f1 + featT_ref[2 * j + 1:2 * j + 2, :]
    out_ref[3:4, :] = f0
    out_ref[4:5, :] = f1
    out_ref[5:8, :] = jnp.zeros((3, out_ref.shape[1]), jnp.float32)


def _run_table(xyzT, featT, interpret=False):
    cn = min(2048, N)
    return pl.pallas_call(
        _table_body,
        grid=(N // cn,),
        in_specs=[
            pl.BlockSpec((3, cn), lambda i: (0, i)),
            pl.BlockSpec((C_IN, cn), lambda i: (0, i)),
        ],
        out_specs=pl.BlockSpec((8, cn), lambda i: (0, i)),
        out_shape=jax.ShapeDtypeStruct((8, N), jnp.float32),
        interpret=interpret,
    )(xyzT, featT)


def _nn_body(cen_ref, supT_ref, pw_ref, pi_ref):
    nchunk = NB // SK
    bpb = CPB // CBLK
    s0 = (pl.program_id(0) // bpb) * NB
    cen = cen_ref[...]
    cx = cen[:, 0:1]
    cy = cen[:, 1:2]
    cz = cen[:, 2:3]
    inf = jnp.full((CBLK, 1), jnp.inf, jnp.float32)
    zi = jnp.zeros((CBLK, 1), jnp.int32)
    d1, d2v, d3 = inf, inf, inf
    a1, a2v, a3 = zi, zi, zi
    lane = lax.broadcasted_iota(jnp.int32, (CBLK, SK), 1)
    for ci in range(nchunk):
        sx = supT_ref[0, 0:1, pl.ds(ci * SK, SK)]
        sy = supT_ref[0, 1:2, pl.ds(ci * SK, SK)]
        sz = supT_ref[0, 2:3, pl.ds(ci * SK, SK)]
        dx = cx - sx
        dy = cy - sy
        dz = cz - sz
        dd = dx * dx + dy * dy + dz * dz
        dd = jnp.where(dd <= jnp.float32(RAD2), dd, jnp.float32(BIG))
        idx = lane + ci * SK
        for k in range(3):
            m = jnp.min(dd, axis=1, keepdims=True)
            am = jnp.min(jnp.where(dd == m, idx, N), axis=1, keepdims=True)
            if k < 2:
                dd = jnp.where(idx == am, jnp.float32(TAKEN), dd)
            lt1 = m < d1
            lt2 = m < d2v
            lt3 = m < d3
            nd1 = jnp.where(lt1, m, d1)
            na1 = jnp.where(lt1, am, a1)
            nd2 = jnp.where(lt1, d1, jnp.where(lt2, m, d2v))
            na2 = jnp.where(lt1, a1, jnp.where(lt2, am, a2v))
            nd3 = jnp.where(lt2, d2v, jnp.where(lt3, m, d3))
            na3 = jnp.where(lt2, a2v, jnp.where(lt3, am, a3))
            d1, d2v, d3, a1, a2v, a3 = nd1, nd2, nd3, na1, na2, na3
    s1 = jnp.minimum(jnp.sqrt(jnp.maximum(d1, 0.0)), jnp.float32(1e8))
    s2 = jnp.minimum(jnp.sqrt(jnp.maximum(d2v, 0.0)), jnp.float32(1e8))
    s3 = jnp.minimum(jnp.sqrt(jnp.maximum(d3, 0.0)), jnp.float32(1e8))
    r1 = 1.0 / (s1 + jnp.float32(1e-8))
    r2 = 1.0 / (s2 + jnp.float32(1e-8))
    r3 = 1.0 / (s3 + jnp.float32(1e-8))
    norm = jnp.maximum(r1 + r2 + r3, jnp.float32(1e-8))
    w1 = r1 / norm
    w2 = r2 / norm
    w3 = r3 / norm
    keep = jnp.where(d1 > jnp.float32(1e20), 0.0, 1.0)
    zf = jnp.zeros((CBLK, 1), jnp.float32)
    pw_ref[...] = jnp.concatenate([w1, w2, w3, cx, cy, cz, keep, zf], axis=1)
    pi_ref[...] = jnp.concatenate(
        [a1 + s0, a2v + s0, a3 + s0, zi, zi, zi, zi, zi], axis=1)


def _run_nn(cflat, supT, interpret=False):
    nblk = CEN // CBLK
    bpb = CPB // CBLK
    return pl.pallas_call(
        _nn_body,
        grid=(nblk,),
        in_specs=[
            pl.BlockSpec((CBLK, 3), lambda i: (i, 0)),
            pl.BlockSpec((1, 3, NB), lambda i: (i // bpb, 0, 0)),
        ],
        out_specs=[
            pl.BlockSpec((CBLK, 8), lambda i: (i, 0)),
            pl.BlockSpec((CBLK, 8), lambda i: (i, 0)),
        ],
        out_shape=[
            jax.ShapeDtypeStruct((CEN, 8), jnp.float32),
            jax.ShapeDtypeStruct((CEN, 8), jnp.int32),
        ],
        interpret=interpret,
    )(cflat, supT)


SC_ROWS = SC_CHUNK // 128


def _sc_assemble(tabs, pws, pis):
    mesh = plsc.VectorSubcoreMesh(core_axis_name="c", subcore_axis_name="s")
    per_w = CEN // NWORK
    nch = per_w // SC_CHUNK

    @functools.partial(
        pl.kernel, mesh=mesh,
        out_type=[jax.ShapeDtypeStruct((CEN,), jnp.float32)
                  for _ in range(11)],
        scratch_types=(
            [pltpu.VMEM((SC_CHUNK,), jnp.float32) for _ in range(7)]
            + [pltpu.VMEM((SC_ROWS, 128), jnp.int32) for _ in range(3)]
            + [pltpu.VMEM((SC_CHUNK,), jnp.float32) for _ in range(15)]
            + [pltpu.VMEM((SC_CHUNK,), jnp.float32) for _ in range(11)]
            + [pltpu.SemaphoreType.DMA]
        ),
    )
    def k(*refs):
        tab_h = refs[0:5]
        pw_h = refs[5:12]
        pi_h = refs[12:15]
        out_h = refs[15:26]
        pw_v = refs[26:33]
        pi_v = refs[33:36]
        g_v = refs[36:51]
        nf_v = refs[51:62]
        sem = refs[62]
        wid = lax.axis_index("s") * 2 + lax.axis_index("c")
        for ch in range(nch):
            base = pl.multiple_of(wid * per_w + ch * SC_CHUNK, 8)
            for f in range(7):
                pltpu.sync_copy(pw_h[f].at[pl.ds(base, SC_CHUNK)], pw_v[f])
            for f in range(3):
                for j in range(SC_ROWS):
                    pltpu.sync_copy(
                        pi_h[f].at[pl.ds(base + j * 128, 128)],
                        pi_v[f].at[j])
            copies = []
            for kk in range(3):
                for f in range(5):
                    for j in range(SC_ROWS):
                        copies.append(pltpu.async_copy(
                            tab_h[f].at[pi_v[kk].at[j]],
                            g_v[kk * 5 + f].at[pl.ds(j * 128, 128)],
                            sem))
            for c in copies:
                c.wait()

            def body(i, carry):
                sl = pl.ds(i * 16, 16)
                w3v = (pw_v[0][sl], pw_v[1][sl], pw_v[2][sl])
                ccx = pw_v[3][sl]
                ccy = pw_v[4][sl]
                ccz = pw_v[5][sl]
                keep = pw_v[6][sl]
                it0 = jnp.zeros((16,), jnp.float32)
                it1 = jnp.zeros((16,), jnp.float32)
                for kk in range(3):
                    gx = g_v[kk * 5 + 0][sl]
                    gy = g_v[kk * 5 + 1][sl]
                    gz = g_v[kk * 5 + 2][sl]
                    gf0 = g_v[kk * 5 + 3][sl]
                    gf1 = g_v[kk * 5 + 4][sl]
                    it0 = it0 + w3v[kk] * gf0
                    it1 = it1 + w3v[kk] * gf1
                    nf_v[2 + 3 * kk][sl] = (ccx - gx) * keep
                    nf_v[3 + 3 * kk][sl] = (ccy - gy) * keep
                    nf_v[4 + 3 * kk][sl] = (ccz - gz) * keep
                nf_v[0][sl] = it0 * keep
                nf_v[1][sl] = it1 * keep
                return carry

            lax.fori_loop(0, SC_CHUNK // 16, body, 0)
            for f in range(11):
                pltpu.sync_copy(nf_v[f], out_h[f].at[pl.ds(base, SC_CHUNK)])

    return k(*tabs, *pws, *pis)


def _mlp_a_body(nf_ref, W1_ref, g1_ref, b1_ref, out_ref):
    RB = min(512, M)
    w1 = W1_ref[0]
    g1v = g1_ref[0]
    b1v = b1_ref[0]
    s = jnp.zeros((1, C_AGG), jnp.float32)
    s2 = jnp.zeros((1, C_AGG), jnp.float32)
    for rb in range(M // RB):
        x = nf_ref[0, pl.ds(rb * RB, RB), :]
        h = jnp.dot(x, w1, preferred_element_type=jnp.float32)
        s = s + jnp.sum(h, axis=0, keepdims=True)
        s2 = s2 + jnp.sum(h * h, axis=0, keepdims=True)
    mu = s / M
    var = s2 / M - mu * mu
    scale = g1v / jnp.sqrt(var + jnp.float32(1e-5))
    shift = b1v - mu * scale
    for rb in range(M // RB):
        x = nf_ref[0, pl.ds(rb * RB, RB), :]
        h = jnp.dot(x, w1, preferred_element_type=jnp.float32)
        out_ref[0, pl.ds(rb * RB, RB), :] = jnp.maximum(h * scale + shift,
                                                        0.0)


def _mlp_b1_body(h_ref, W2_ref, acc_ref):
    xs = [h_ref[v] for v in range(TOTAL_VOX)]
    x = jnp.concatenate(xs, axis=1)
    acc_ref[...] = jnp.dot(x, W2_ref[...],
                           preferred_element_type=jnp.float32)


def _mlp_b2_body(acc_ref, g2_ref, b2_ref, out_ref):
    RB = min(512, M)
    s = jnp.zeros((1, POST), jnp.float32)
    s2 = jnp.zeros((1, POST), jnp.float32)
    for rb in range(M // RB):
        part = acc_ref[pl.ds(rb * RB, RB), :]
        s = s + jnp.sum(part, axis=0, keepdims=True)
        s2 = s2 + jnp.sum(part * part, axis=0, keepdims=True)
    mu = s / M
    var = s2 / M - mu * mu
    scale = g2_ref[...] / jnp.sqrt(var + jnp.float32(1e-5))
    shift = b2_ref[...] - mu * scale
    for rb in range(M // RB):
        h = acc_ref[pl.ds(rb * RB, RB), :]
        out_ref[pl.ds(rb * RB, RB), :] = jnp.maximum(h * scale + shift, 0.0)


def _run_mlp(nfT, W1, g1, b1, W2, g2, b2, interpret=False):
    h = pl.pallas_call(
        _mlp_a_body,
        grid=(TOTAL_VOX,),
        in_specs=[
            pl.BlockSpec((1, M, CR), lambda v: (v, 0, 0)),
            pl.BlockSpec((1, CR, C_AGG), lambda v: (v, 0, 0)),
            pl.BlockSpec((1, 1, C_AGG), lambda v: (v, 0, 0)),
            pl.BlockSpec((1, 1, C_AGG), lambda v: (v, 0, 0)),
        ],
        out_specs=pl.BlockSpec((1, M, C_AGG), lambda v: (v, 0, 0)),
        out_shape=jax.ShapeDtypeStruct((TOTAL_VOX, M, C_AGG), jnp.float32),
        interpret=interpret,
    )(nfT, W1, g1.reshape(TOTAL_VOX, 1, C_AGG), b1.reshape(TOTAL_VOX, 1, C_AGG))
    RB = min(512, M)
    acc = pl.pallas_call(
        _mlp_b1_body,
        grid=(M // RB,),
        in_specs=[
            pl.BlockSpec((TOTAL_VOX, RB, C_AGG), lambda r: (0, r, 0)),
            pl.BlockSpec((TOTAL_VOX * C_AGG, POST), lambda r: (0, 0)),
        ],
        out_specs=pl.BlockSpec((RB, POST), lambda r: (r, 0)),
        out_shape=jax.ShapeDtypeStruct((M, POST), jnp.float32),
        interpret=interpret,
    )(h, W2)
    return pl.pallas_call(
        _mlp_b2_body,
        in_specs=[
            pl.BlockSpec((M, POST), lambda: (0, 0)),
            pl.BlockSpec((1, POST), lambda: (0, 0)),
            pl.BlockSpec((1, POST), lambda: (0, 0)),
        ],
        out_specs=pl.BlockSpec((M, POST), lambda: (0, 0)),
        out_shape=jax.ShapeDtypeStruct((M, POST), jnp.float32),
        interpret=interpret,
    )(acc, g2.reshape(1, POST), b2.reshape(1, POST))


def _impl(support_xyz, support_features, new_xyz, W1, g1, b1, W2, g2, b2,
          interpret=False):
    centers = new_xyz[:, None, :] + _voxel_offs()[None, :, :]
    cflat = centers.reshape(-1, 3)
    supT = support_xyz.reshape(B, NB, 3).transpose(0, 2, 1)
    tab = _run_table(support_xyz.T, support_features.T, interpret=interpret)
    pw, pi = _run_nn(cflat, supT, interpret=interpret)
    pwT = pw.T
    piT = pi.T
    nfs = _sc_assemble([tab[f] for f in range(5)],
                       [pwT[f] for f in range(7)],
                       [piT[f] for f in range(3)])
    nfT = jnp.stack(nfs)
    nf3 = nfT.T.reshape(M, TOTAL_VOX, CR).transpose(1, 0, 2)
    return _run_mlp(nf3, W1, g1, b1, W2, g2, b2, interpret=interpret)


def kernel(support_xyz, support_features, batch_num_xyzs, new_xyz,
           batch_num_new_xyzs, W1, g1, b1, W2, g2, b2):
    return _impl(support_xyz, support_features, new_xyz, W1, g1, b1, W2, g2,
                 b2)


# speedup vs baseline: 4.3545x; 1.0309x over previous
import functools

import jax
import jax.numpy as jnp
from jax import lax
from jax.experimental import pallas as pl
from jax.experimental.pallas import tpu as pltpu
from jax.experimental.pallas import tpu_sc as plsc

N = 16384
M = 4096
B = 2
C_IN = 32
NUM_REDUCED = 2
TOTAL_VOX = 27
R = 1.2
MULT = 2.0
C_AGG = 32
POST = 128
CR = NUM_REDUCED + 9

NB = N // B
CEN = M * TOTAL_VOX
CPB = CEN // B
RAD2 = (R * MULT) ** 2
BIG = 1e30
TAKEN = 2e30

CBLK = 128
SK = 512

NWORK = 32
SC_CHUNK = 1152


def _voxel_offs():
    g = jnp.array([-2.0 * R / 3.0, 0.0, 2.0 * R / 3.0], dtype=jnp.float32)
    xx, yy, zz = jnp.meshgrid(g, g, g, indexing='ij')
    return jnp.stack([xx, yy, zz], axis=-1).reshape(-1, 3)


def _table_body(xyzT_ref, featT_ref, out_ref):
    out_ref[0:3, :] = xyzT_ref[...]
    f0 = featT_ref[0:1, :]
    f1 = featT_ref[1:2, :]
    for j in range(1, C_IN // NUM_REDUCED):
        f0 = f0 + featT_ref[2 * j:2 * j + 1, :]
        f1 = f1 + featT_ref[2 * j + 1:2 * j + 2, :]
    out_ref[3:4, :] = f0
    out_ref[4:5, :] = f1
    out_ref[5:8, :] = jnp.zeros((3, out_ref.shape[1]), jnp.float32)


def _run_table(xyzT, featT, interpret=False):
    cn = min(2048, N)
    return pl.pallas_call(
        _table_body,
        grid=(N // cn,),
        in_specs=[
            pl.BlockSpec((3, cn), lambda i: (0, i)),
            pl.BlockSpec((C_IN, cn), lambda i: (0, i)),
        ],
        out_specs=pl.BlockSpec((8, cn), lambda i: (0, i)),
        out_shape=jax.ShapeDtypeStruct((8, N), jnp.float32),
        interpret=interpret,
    )(xyzT, featT)


def _nn_body(cen_ref, supT_ref, pw_ref, pi_ref):
    nchunk = NB // SK
    mpb = M // CBLK
    qpb = M // B // CBLK
    s0 = ((pl.program_id(0) % mpb) // qpb) * NB
    cen = cen_ref[...]
    cx = cen[:, 0:1]
    cy = cen[:, 1:2]
    cz = cen[:, 2:3]
    inf = jnp.full((CBLK, 1), jnp.inf, jnp.float32)
    zi = jnp.zeros((CBLK, 1), jnp.int32)
    d1, d2v, d3 = inf, inf, inf
    a1, a2v, a3 = zi, zi, zi
    lane = lax.broadcasted_iota(jnp.int32, (CBLK, SK), 1)
    for ci in range(nchunk):
        sx = supT_ref[0, 0:1, pl.ds(ci * SK, SK)]
        sy = supT_ref[0, 1:2, pl.ds(ci * SK, SK)]
        sz = supT_ref[0, 2:3, pl.ds(ci * SK, SK)]
        dx = cx - sx
        dy = cy - sy
        dz = cz - sz
        dd = dx * dx + dy * dy + dz * dz
        dd = jnp.where(dd <= jnp.float32(RAD2), dd, jnp.float32(BIG))
        idx = lane + ci * SK
        for k in range(3):
            m = jnp.min(dd, axis=1, keepdims=True)
            am = jnp.min(jnp.where(dd == m, idx, N), axis=1, keepdims=True)
            if k < 2:
                dd = jnp.where(idx == am, jnp.float32(TAKEN), dd)
            lt1 = m < d1
            lt2 = m < d2v
            lt3 = m < d3
            nd1 = jnp.where(lt1, m, d1)
            na1 = jnp.where(lt1, am, a1)
            nd2 = jnp.where(lt1, d1, jnp.where(lt2, m, d2v))
            na2 = jnp.where(lt1, a1, jnp.where(lt2, am, a2v))
            nd3 = jnp.where(lt2, d2v, jnp.where(lt3, m, d3))
            na3 = jnp.where(lt2, a2v, jnp.where(lt3, am, a3))
            d1, d2v, d3, a1, a2v, a3 = nd1, nd2, nd3, na1, na2, na3
    s1 = jnp.minimum(jnp.sqrt(jnp.maximum(d1, 0.0)), jnp.float32(1e8))
    s2 = jnp.minimum(jnp.sqrt(jnp.maximum(d2v, 0.0)), jnp.float32(1e8))
    s3 = jnp.minimum(jnp.sqrt(jnp.maximum(d3, 0.0)), jnp.float32(1e8))
    r1 = 1.0 / (s1 + jnp.float32(1e-8))
    r2 = 1.0 / (s2 + jnp.float32(1e-8))
    r3 = 1.0 / (s3 + jnp.float32(1e-8))
    norm = jnp.maximum(r1 + r2 + r3, jnp.float32(1e-8))
    w1 = r1 / norm
    w2 = r2 / norm
    w3 = r3 / norm
    keep = jnp.where(d1 > jnp.float32(1e20), 0.0, 1.0)
    zf = jnp.zeros((CBLK, 1), jnp.float32)
    pw_ref[...] = jnp.transpose(
        jnp.concatenate([w1, w2, w3, cx, cy, cz, keep, zf], axis=1))
    pi_ref[...] = jnp.transpose(jnp.concatenate(
        [a1 + s0, a2v + s0, a3 + s0, zi, zi, zi, zi, zi], axis=1))


def _run_nn(cflat, supT, interpret=False):
    nblk = CEN // CBLK
    mpb = M // CBLK
    qpb = M // B // CBLK
    return pl.pallas_call(
        _nn_body,
        grid=(nblk,),
        in_specs=[
            pl.BlockSpec((CBLK, 3), lambda i: (i, 0)),
            pl.BlockSpec((1, 3, NB), lambda i: ((i % mpb) // qpb, 0, 0)),
        ],
        out_specs=[
            pl.BlockSpec((8, CBLK), lambda i: (0, i)),
            pl.BlockSpec((8, CBLK), lambda i: (0, i)),
        ],
        out_shape=[
            jax.ShapeDtypeStruct((8, CEN), jnp.float32),
            jax.ShapeDtypeStruct((8, CEN), jnp.int32),
        ],
        interpret=interpret,
    )(cflat, supT)


SC_ROWS = SC_CHUNK // 128


def _sc_assemble(tabs, pws, pis):
    mesh = plsc.VectorSubcoreMesh(core_axis_name="c", subcore_axis_name="s")
    per_w = CEN // NWORK
    nch = per_w // SC_CHUNK

    @functools.partial(
        pl.kernel, mesh=mesh,
        out_type=[jax.ShapeDtypeStruct((CEN,), jnp.float32)
                  for _ in range(11)],
        scratch_types=(
            [pltpu.VMEM((SC_CHUNK,), jnp.float32) for _ in range(7)]
            + [pltpu.VMEM((SC_ROWS, 128), jnp.int32) for _ in range(3)]
            + [pltpu.VMEM((SC_CHUNK,), jnp.float32) for _ in range(15)]
            + [pltpu.VMEM((SC_CHUNK,), jnp.float32) for _ in range(11)]
            + [pltpu.SemaphoreType.DMA]
        ),
    )
    def k(*refs):
        tab_h = refs[0:5]
        pw_h = refs[5:12]
        pi_h = refs[12:15]
        out_h = refs[15:26]
        pw_v = refs[26:33]
        pi_v = refs[33:36]
        g_v = refs[36:51]
        nf_v = refs[51:62]
        sem = refs[62]
        wid = lax.axis_index("s") * 2 + lax.axis_index("c")
        for ch in range(nch):
            base = pl.multiple_of(wid * per_w + ch * SC_CHUNK, 8)
            for f in range(7):
                pltpu.sync_copy(pw_h[f].at[pl.ds(base, SC_CHUNK)], pw_v[f])
            for f in range(3):
                for j in range(SC_ROWS):
                    pltpu.sync_copy(
                        pi_h[f].at[pl.ds(base + j * 128, 128)],
                        pi_v[f].at[j])
            copies = []
            for kk in range(3):
                for f in range(5):
                    for j in range(SC_ROWS):
                        copies.append(pltpu.async_copy(
                            tab_h[f].at[pi_v[kk].at[j]],
                            g_v[kk * 5 + f].at[pl.ds(j * 128, 128)],
                            sem))
            for c in copies:
                c.wait()

            def body(i, carry):
                sl = pl.ds(i * 16, 16)
                w3v = (pw_v[0][sl], pw_v[1][sl], pw_v[2][sl])
                ccx = pw_v[3][sl]
                ccy = pw_v[4][sl]
                ccz = pw_v[5][sl]
                keep = pw_v[6][sl]
                it0 = jnp.zeros((16,), jnp.float32)
                it1 = jnp.zeros((16,), jnp.float32)
                for kk in range(3):
                    gx = g_v[kk * 5 + 0][sl]
                    gy = g_v[kk * 5 + 1][sl]
                    gz = g_v[kk * 5 + 2][sl]
                    gf0 = g_v[kk * 5 + 3][sl]
                    gf1 = g_v[kk * 5 + 4][sl]
                    it0 = it0 + w3v[kk] * gf0
                    it1 = it1 + w3v[kk] * gf1
                    nf_v[2 + 3 * kk][sl] = (ccx - gx) * keep
                    nf_v[3 + 3 * kk][sl] = (ccy - gy) * keep
                    nf_v[4 + 3 * kk][sl] = (ccz - gz) * keep
                nf_v[0][sl] = it0 * keep
                nf_v[1][sl] = it1 * keep
                return carry

            lax.fori_loop(0, SC_CHUNK // 16, body, 0)
            for f in range(11):
                pltpu.sync_copy(nf_v[f], out_h[f].at[pl.ds(base, SC_CHUNK)])

    return k(*tabs, *pws, *pis)


def _mlp_a_body(nf_ref, W1_ref, g1_ref, b1_ref, out_ref):
    RB = min(512, M)
    w1 = W1_ref[0]
    g1v = g1_ref[0]
    b1v = b1_ref[0]
    s = jnp.zeros((1, C_AGG), jnp.float32)
    s2 = jnp.zeros((1, C_AGG), jnp.float32)
    dn = (((0,), (0,)), ((), ()))
    for rb in range(M // RB):
        xT = nf_ref[0, :, pl.ds(rb * RB, RB)]
        h = lax.dot_general(xT, w1, dn, preferred_element_type=jnp.float32)
        s = s + jnp.sum(h, axis=0, keepdims=True)
        s2 = s2 + jnp.sum(h * h, axis=0, keepdims=True)
    mu = s / M
    var = s2 / M - mu * mu
    scale = g1v / jnp.sqrt(var + jnp.float32(1e-5))
    shift = b1v - mu * scale
    for rb in range(M // RB):
        xT = nf_ref[0, :, pl.ds(rb * RB, RB)]
        h = lax.dot_general(xT, w1, dn, preferred_element_type=jnp.float32)
        out_ref[0, pl.ds(rb * RB, RB), :] = jnp.maximum(h * scale + shift,
                                                        0.0)


def _mlp_b1_body(h_ref, W2_ref, acc_ref):
    xs = [h_ref[v] for v in range(TOTAL_VOX)]
    x = jnp.concatenate(xs, axis=1)
    acc_ref[...] = jnp.dot(x, W2_ref[...],
                           preferred_element_type=jnp.float32)


def _mlp_b2_body(acc_ref, g2_ref, b2_ref, out_ref):
    RB = min(512, M)
    s = jnp.zeros((1, POST), jnp.float32)
    s2 = jnp.zeros((1, POST), jnp.float32)
    for rb in range(M // RB):
        part = acc_ref[pl.ds(rb * RB, RB), :]
        s = s + jnp.sum(part, axis=0, keepdims=True)
        s2 = s2 + jnp.sum(part * part, axis=0, keepdims=True)
    mu = s / M
    var = s2 / M - mu * mu
    scale = g2_ref[...] / jnp.sqrt(var + jnp.float32(1e-5))
    shift = b2_ref[...] - mu * scale
    for rb in range(M // RB):
        h = acc_ref[pl.ds(rb * RB, RB), :]
        out_ref[pl.ds(rb * RB, RB), :] = jnp.maximum(h * scale + shift, 0.0)


def _run_mlp(nfT, W1, g1, b1, W2, g2, b2, interpret=False):
    h = pl.pallas_call(
        _mlp_a_body,
        grid=(TOTAL_VOX,),
        in_specs=[
            pl.BlockSpec((1, CR, M), lambda v: (v, 0, 0)),
            pl.BlockSpec((1, CR, C_AGG), lambda v: (v, 0, 0)),
            pl.BlockSpec((1, 1, C_AGG), lambda v: (v, 0, 0)),
            pl.BlockSpec((1, 1, C_AGG), lambda v: (v, 0, 0)),
        ],
        out_specs=pl.BlockSpec((1, M, C_AGG), lambda v: (v, 0, 0)),
        out_shape=jax.ShapeDtypeStruct((TOTAL_VOX, M, C_AGG), jnp.float32),
        interpret=interpret,
    )(nfT, W1, g1.reshape(TOTAL_VOX, 1, C_AGG), b1.reshape(TOTAL_VOX, 1, C_AGG))
    RB = min(512, M)
    acc = pl.pallas_call(
        _mlp_b1_body,
        grid=(M // RB,),
        in_specs=[
            pl.BlockSpec((TOTAL_VOX, RB, C_AGG), lambda r: (0, r, 0)),
            pl.BlockSpec((TOTAL_VOX * C_AGG, POST), lambda r: (0, 0)),
        ],
        out_specs=pl.BlockSpec((RB, POST), lambda r: (r, 0)),
        out_shape=jax.ShapeDtypeStruct((M, POST), jnp.float32),
        interpret=interpret,
    )(h, W2)
    return pl.pallas_call(
        _mlp_b2_body,
        in_specs=[
            pl.BlockSpec((M, POST), lambda: (0, 0)),
            pl.BlockSpec((1, POST), lambda: (0, 0)),
            pl.BlockSpec((1, POST), lambda: (0, 0)),
        ],
        out_specs=pl.BlockSpec((M, POST), lambda: (0, 0)),
        out_shape=jax.ShapeDtypeStruct((M, POST), jnp.float32),
        interpret=interpret,
    )(acc, g2.reshape(1, POST), b2.reshape(1, POST))


def _impl(support_xyz, support_features, new_xyz, W1, g1, b1, W2, g2, b2,
          interpret=False):
    centers = _voxel_offs()[:, None, :] + new_xyz[None, :, :]
    cflat = centers.reshape(-1, 3)
    supT = support_xyz.reshape(B, NB, 3).transpose(0, 2, 1)
    tab = _run_table(support_xyz.T, support_features.T, interpret=interpret)
    pw, pi = _run_nn(cflat, supT, interpret=interpret)
    nfs = _sc_assemble([tab[f] for f in range(5)],
                       [pw[f] for f in range(7)],
                       [pi[f] for f in range(3)])
    nf3 = jnp.stack([x.reshape(TOTAL_VOX, M) for x in nfs], axis=1)
    return _run_mlp(nf3, W1, g1, b1, W2, g2, b2, interpret=interpret)


def kernel(support_xyz, support_features, batch_num_xyzs, new_xyz,
           batch_num_new_xyzs, W1, g1, b1, W2, g2, b2):
    return _impl(support_xyz, support_features, new_xyz, W1, g1, b1, W2, g2,
                 b2)


# speedup vs baseline: 7.8850x; 1.8108x over previous
import functools

import jax
import jax.numpy as jnp
from jax import lax
from jax.experimental import pallas as pl
from jax.experimental.pallas import tpu as pltpu
from jax.experimental.pallas import tpu_sc as plsc

N = 16384
M = 4096
B = 2
C_IN = 32
NUM_REDUCED = 2
TOTAL_VOX = 27
R = 1.2
MULT = 2.0
C_AGG = 32
POST = 128
CR = NUM_REDUCED + 9

NB = N // B
CEN = M * TOTAL_VOX
CPB = CEN // B
RAD2 = (R * MULT) ** 2
BIG = 1e30
TAKEN = 2e30

CBLK = 256
SK = 256

NWORK = 32
SC_CHUNK = 1152


def _voxel_offs():
    g = jnp.array([-2.0 * R / 3.0, 0.0, 2.0 * R / 3.0], dtype=jnp.float32)
    xx, yy, zz = jnp.meshgrid(g, g, g, indexing='ij')
    return jnp.stack([xx, yy, zz], axis=-1).reshape(-1, 3)


def _table_body(xyzT_ref, featT_ref, out_ref):
    out_ref[0:3, :] = xyzT_ref[...]
    f0 = featT_ref[0:1, :]
    f1 = featT_ref[1:2, :]
    for j in range(1, C_IN // NUM_REDUCED):
        f0 = f0 + featT_ref[2 * j:2 * j + 1, :]
        f1 = f1 + featT_ref[2 * j + 1:2 * j + 2, :]
    out_ref[3:4, :] = f0
    out_ref[4:5, :] = f1
    out_ref[5:8, :] = jnp.zeros((3, out_ref.shape[1]), jnp.float32)


def _run_table(xyzT, featT, interpret=False):
    cn = min(2048, N)
    return pl.pallas_call(
        _table_body,
        grid=(N // cn,),
        in_specs=[
            pl.BlockSpec((3, cn), lambda i: (0, i)),
            pl.BlockSpec((C_IN, cn), lambda i: (0, i)),
        ],
        out_specs=pl.BlockSpec((8, cn), lambda i: (0, i)),
        out_shape=jax.ShapeDtypeStruct((8, N), jnp.float32),
        interpret=interpret,
    )(xyzT, featT)


def _nn_body(cenT_ref, sup_ref, pw_ref, pi_ref):
    nchunk = NB // SK
    mpb = M // CBLK
    qpb = M // B // CBLK
    s0 = ((pl.program_id(0) % mpb) // qpb) * NB
    cx = cenT_ref[0:1, :]
    cy = cenT_ref[1:2, :]
    cz = cenT_ref[2:3, :]
    inf = jnp.full((1, CBLK), jnp.inf, jnp.float32)
    zi = jnp.zeros((1, CBLK), jnp.int32)
    d1, d2v, d3 = inf, inf, inf
    a1, a2v, a3 = zi, zi, zi
    lane = lax.broadcasted_iota(jnp.int32, (SK, CBLK), 0)
    for ci in range(nchunk):
        sx = sup_ref[0, pl.ds(ci * SK, SK), 0:1]
        sy = sup_ref[0, pl.ds(ci * SK, SK), 1:2]
        sz = sup_ref[0, pl.ds(ci * SK, SK), 2:3]
        dx = cx - sx
        dy = cy - sy
        dz = cz - sz
        dd = dx * dx + dy * dy + dz * dz
        dd = jnp.where(dd <= jnp.float32(RAD2), dd, jnp.float32(BIG))
        idx = lane + ci * SK
        for k in range(3):
            m = jnp.min(dd, axis=0, keepdims=True)
            am = jnp.min(jnp.where(dd == m, idx, N), axis=0, keepdims=True)
            if k < 2:
                dd = jnp.where(idx == am, jnp.float32(TAKEN), dd)
            lt1 = m < d1
            lt2 = m < d2v
            lt3 = m < d3
            nd1 = jnp.where(lt1, m, d1)
            na1 = jnp.where(lt1, am, a1)
            nd2 = jnp.where(lt1, d1, jnp.where(lt2, m, d2v))
            na2 = jnp.where(lt1, a1, jnp.where(lt2, am, a2v))
            nd3 = jnp.where(lt2, d2v, jnp.where(lt3, m, d3))
            na3 = jnp.where(lt2, a2v, jnp.where(lt3, am, a3))
            d1, d2v, d3, a1, a2v, a3 = nd1, nd2, nd3, na1, na2, na3
    s1 = jnp.minimum(jnp.sqrt(jnp.maximum(d1, 0.0)), jnp.float32(1e8))
    s2 = jnp.minimum(jnp.sqrt(jnp.maximum(d2v, 0.0)), jnp.float32(1e8))
    s3 = jnp.minimum(jnp.sqrt(jnp.maximum(d3, 0.0)), jnp.float32(1e8))
    r1 = 1.0 / (s1 + jnp.float32(1e-8))
    r2 = 1.0 / (s2 + jnp.float32(1e-8))
    r3 = 1.0 / (s3 + jnp.float32(1e-8))
    norm = jnp.maximum(r1 + r2 + r3, jnp.float32(1e-8))
    w1 = r1 / norm
    w2 = r2 / norm
    w3 = r3 / norm
    keep = jnp.where(d1 > jnp.float32(1e20), 0.0, 1.0)
    zf = jnp.zeros((1, CBLK), jnp.float32)
    pw_ref[...] = jnp.concatenate([w1, w2, w3, cx, cy, cz, keep, zf], axis=0)
    pi_ref[...] = jnp.concatenate(
        [a1 + s0, a2v + s0, a3 + s0, zi, zi, zi, zi, zi], axis=0)


def _run_nn(cflatT, sup3, interpret=False):
    nblk = CEN // CBLK
    mpb = M // CBLK
    qpb = M // B // CBLK
    return pl.pallas_call(
        _nn_body,
        grid=(nblk,),
        in_specs=[
            pl.BlockSpec((3, CBLK), lambda i: (0, i)),
            pl.BlockSpec((1, NB, 3), lambda i: ((i % mpb) // qpb, 0, 0)),
        ],
        out_specs=[
            pl.BlockSpec((8, CBLK), lambda i: (0, i)),
            pl.BlockSpec((8, CBLK), lambda i: (0, i)),
        ],
        out_shape=[
            jax.ShapeDtypeStruct((8, CEN), jnp.float32),
            jax.ShapeDtypeStruct((8, CEN), jnp.int32),
        ],
        interpret=interpret,
    )(cflatT, sup3)


SC_ROWS = SC_CHUNK // 128


def _sc_assemble(tabs, pws, pis):
    mesh = plsc.VectorSubcoreMesh(core_axis_name="c", subcore_axis_name="s")
    per_w = CEN // NWORK
    nch = per_w // SC_CHUNK

    @functools.partial(
        pl.kernel, mesh=mesh,
        out_type=[jax.ShapeDtypeStruct((CEN,), jnp.float32)
                  for _ in range(11)],
        scratch_types=(
            [pltpu.VMEM((SC_CHUNK,), jnp.float32) for _ in range(7)]
            + [pltpu.VMEM((SC_ROWS, 128), jnp.int32) for _ in range(3)]
            + [pltpu.VMEM((SC_CHUNK,), jnp.float32) for _ in range(15)]
            + [pltpu.VMEM((SC_CHUNK,), jnp.float32) for _ in range(11)]
            + [pltpu.SemaphoreType.DMA]
        ),
    )
    def k(*refs):
        tab_h = refs[0:5]
        pw_h = refs[5:12]
        pi_h = refs[12:15]
        out_h = refs[15:26]
        pw_v = refs[26:33]
        pi_v = refs[33:36]
        g_v = refs[36:51]
        nf_v = refs[51:62]
        sem = refs[62]
        wid = lax.axis_index("s") * 2 + lax.axis_index("c")
        for ch in range(nch):
            base = pl.multiple_of(wid * per_w + ch * SC_CHUNK, 8)
            for f in range(7):
                pltpu.sync_copy(pw_h[f].at[pl.ds(base, SC_CHUNK)], pw_v[f])
            for f in range(3):
                for j in range(SC_ROWS):
                    pltpu.sync_copy(
                        pi_h[f].at[pl.ds(base + j * 128, 128)],
                        pi_v[f].at[j])
            copies = []
            for kk in range(3):
                for f in range(5):
                    for j in range(SC_ROWS):
                        copies.append(pltpu.async_copy(
                            tab_h[f].at[pi_v[kk].at[j]],
                            g_v[kk * 5 + f].at[pl.ds(j * 128, 128)],
                            sem))
            for c in copies:
                c.wait()

            def body(i, carry):
                sl = pl.ds(i * 16, 16)
                w3v = (pw_v[0][sl], pw_v[1][sl], pw_v[2][sl])
                ccx = pw_v[3][sl]
                ccy = pw_v[4][sl]
                ccz = pw_v[5][sl]
                keep = pw_v[6][sl]
                it0 = jnp.zeros((16,), jnp.float32)
                it1 = jnp.zeros((16,), jnp.float32)
                for kk in range(3):
                    gx = g_v[kk * 5 + 0][sl]
                    gy = g_v[kk * 5 + 1][sl]
                    gz = g_v[kk * 5 + 2][sl]
                    gf0 = g_v[kk * 5 + 3][sl]
                    gf1 = g_v[kk * 5 + 4][sl]
                    it0 = it0 + w3v[kk] * gf0
                    it1 = it1 + w3v[kk] * gf1
                    nf_v[2 + 3 * kk][sl] = (ccx - gx) * keep
                    nf_v[3 + 3 * kk][sl] = (ccy - gy) * keep
                    nf_v[4 + 3 * kk][sl] = (ccz - gz) * keep
                nf_v[0][sl] = it0 * keep
                nf_v[1][sl] = it1 * keep
                return carry

            lax.fori_loop(0, SC_CHUNK // 16, body, 0)
            for f in range(11):
                pltpu.sync_copy(nf_v[f], out_h[f].at[pl.ds(base, SC_CHUNK)])

    return k(*tabs, *pws, *pis)


def _mlp_a_body(nf_ref, W1_ref, g1_ref, b1_ref, out_ref):
    RB = min(512, M)
    w1 = W1_ref[0]
    g1v = g1_ref[0]
    b1v = b1_ref[0]
    s = jnp.zeros((1, C_AGG), jnp.float32)
    s2 = jnp.zeros((1, C_AGG), jnp.float32)
    dn = (((0,), (0,)), ((), ()))
    for rb in range(M // RB):
        xT = nf_ref[0, :, pl.ds(rb * RB, RB)]
        h = lax.dot_general(xT, w1, dn, preferred_element_type=jnp.float32)
        s = s + jnp.sum(h, axis=0, keepdims=True)
        s2 = s2 + jnp.sum(h * h, axis=0, keepdims=True)
    mu = s / M
    var = s2 / M - mu * mu
    scale = g1v / jnp.sqrt(var + jnp.float32(1e-5))
    shift = b1v - mu * scale
    for rb in range(M // RB):
        xT = nf_ref[0, :, pl.ds(rb * RB, RB)]
        h = lax.dot_general(xT, w1, dn, preferred_element_type=jnp.float32)
        out_ref[0, pl.ds(rb * RB, RB), :] = jnp.maximum(h * scale + shift,
                                                        0.0)


def _mlp_b1_body(h_ref, W2_ref, acc_ref):
    xs = [h_ref[v] for v in range(TOTAL_VOX)]
    x = jnp.concatenate(xs, axis=1)
    acc_ref[...] = jnp.dot(x, W2_ref[...],
                           preferred_element_type=jnp.float32)


def _mlp_b2_body(acc_ref, g2_ref, b2_ref, out_ref):
    RB = min(512, M)
    s = jnp.zeros((1, POST), jnp.float32)
    s2 = jnp.zeros((1, POST), jnp.float32)
    for rb in range(M // RB):
        part = acc_ref[pl.ds(rb * RB, RB), :]
        s = s + jnp.sum(part, axis=0, keepdims=True)
        s2 = s2 + jnp.sum(part * part, axis=0, keepdims=True)
    mu = s / M
    var = s2 / M - mu * mu
    scale = g2_ref[...] / jnp.sqrt(var + jnp.float32(1e-5))
    shift = b2_ref[...] - mu * scale
    for rb in range(M // RB):
        h = acc_ref[pl.ds(rb * RB, RB), :]
        out_ref[pl.ds(rb * RB, RB), :] = jnp.maximum(h * scale + shift, 0.0)


def _run_mlp(nfT, W1, g1, b1, W2, g2, b2, interpret=False):
    h = pl.pallas_call(
        _mlp_a_body,
        grid=(TOTAL_VOX,),
        in_specs=[
            pl.BlockSpec((1, CR, M), lambda v: (v, 0, 0)),
            pl.BlockSpec((1, CR, C_AGG), lambda v: (v, 0, 0)),
            pl.BlockSpec((1, 1, C_AGG), lambda v: (v, 0, 0)),
            pl.BlockSpec((1, 1, C_AGG), lambda v: (v, 0, 0)),
        ],
        out_specs=pl.BlockSpec((1, M, C_AGG), lambda v: (v, 0, 0)),
        out_shape=jax.ShapeDtypeStruct((TOTAL_VOX, M, C_AGG), jnp.float32),
        interpret=interpret,
    )(nfT, W1, g1.reshape(TOTAL_VOX, 1, C_AGG), b1.reshape(TOTAL_VOX, 1, C_AGG))
    RB = min(512, M)
    acc = pl.pallas_call(
        _mlp_b1_body,
        grid=(M // RB,),
        in_specs=[
            pl.BlockSpec((TOTAL_VOX, RB, C_AGG), lambda r: (0, r, 0)),
            pl.BlockSpec((TOTAL_VOX * C_AGG, POST), lambda r: (0, 0)),
        ],
        out_specs=pl.BlockSpec((RB, POST), lambda r: (r, 0)),
        out_shape=jax.ShapeDtypeStruct((M, POST), jnp.float32),
        interpret=interpret,
    )(h, W2)
    return pl.pallas_call(
        _mlp_b2_body,
        in_specs=[
            pl.BlockSpec((M, POST), lambda: (0, 0)),
            pl.BlockSpec((1, POST), lambda: (0, 0)),
            pl.BlockSpec((1, POST), lambda: (0, 0)),
        ],
        out_specs=pl.BlockSpec((M, POST), lambda: (0, 0)),
        out_shape=jax.ShapeDtypeStruct((M, POST), jnp.float32),
        interpret=interpret,
    )(acc, g2.reshape(1, POST), b2.reshape(1, POST))


def _impl(support_xyz, support_features, new_xyz, W1, g1, b1, W2, g2, b2,
          interpret=False):
    centers = _voxel_offs()[:, None, :] + new_xyz[None, :, :]
    cflatT = centers.reshape(-1, 3).T
    sup3 = support_xyz.reshape(B, NB, 3)
    tab = _run_table(support_xyz.T, support_features.T, interpret=interpret)
    pw, pi = _run_nn(cflatT, sup3, interpret=interpret)
    nfs = _sc_assemble([tab[f] for f in range(5)],
                       [pw[f] for f in range(7)],
                       [pi[f] for f in range(3)])
    nf3 = jnp.stack([x.reshape(TOTAL_VOX, M) for x in nfs], axis=1)
    return _run_mlp(nf3, W1, g1, b1, W2, g2, b2, interpret=interpret)


def kernel(support_xyz, support_features, batch_num_xyzs, new_xyz,
           batch_num_new_xyzs, W1, g1, b1, W2, g2, b2):
    return _impl(support_xyz, support_features, new_xyz, W1, g1, b1, W2, g2,
                 b2)


# speedup vs baseline: 8.0972x; 1.0269x over previous
import functools

import jax
import jax.numpy as jnp
from jax import lax
from jax.experimental import pallas as pl
from jax.experimental.pallas import tpu as pltpu
from jax.experimental.pallas import tpu_sc as plsc

N = 16384
M = 4096
B = 2
C_IN = 32
NUM_REDUCED = 2
TOTAL_VOX = 27
R = 1.2
MULT = 2.0
C_AGG = 32
POST = 128
CR = NUM_REDUCED + 9

NB = N // B
CEN = M * TOTAL_VOX
CPB = CEN // B
RAD2 = (R * MULT) ** 2
BIG = 1e30
TAKEN = 2e30

CBLK = 512
SK = 256

NWORK = 32
SC_CHUNK = 1152


def _voxel_offs():
    g = jnp.array([-2.0 * R / 3.0, 0.0, 2.0 * R / 3.0], dtype=jnp.float32)
    xx, yy, zz = jnp.meshgrid(g, g, g, indexing='ij')
    return jnp.stack([xx, yy, zz], axis=-1).reshape(-1, 3)


def _table_body(xyzT_ref, featT_ref, out_ref):
    out_ref[0:3, :] = xyzT_ref[...]
    f0 = featT_ref[0:1, :]
    f1 = featT_ref[1:2, :]
    for j in range(1, C_IN // NUM_REDUCED):
        f0 = f0 + featT_ref[2 * j:2 * j + 1, :]
        f1 = f1 + featT_ref[2 * j + 1:2 * j + 2, :]
    out_ref[3:4, :] = f0
    out_ref[4:5, :] = f1
    out_ref[5:8, :] = jnp.zeros((3, out_ref.shape[1]), jnp.float32)


def _run_table(xyzT, featT, interpret=False):
    cn = min(2048, N)
    return pl.pallas_call(
        _table_body,
        grid=(N // cn,),
        in_specs=[
            pl.BlockSpec((3, cn), lambda i: (0, i)),
            pl.BlockSpec((C_IN, cn), lambda i: (0, i)),
        ],
        out_specs=pl.BlockSpec((8, cn), lambda i: (0, i)),
        out_shape=jax.ShapeDtypeStruct((8, N), jnp.float32),
        interpret=interpret,
    )(xyzT, featT)


def _nn_body(cenT_ref, sup_ref, pw_ref, pi_ref):
    nchunk = NB // SK
    mpb = M // CBLK
    qpb = M // B // CBLK
    s0 = ((pl.program_id(0) % mpb) // qpb) * NB
    cx = cenT_ref[0:1, :]
    cy = cenT_ref[1:2, :]
    cz = cenT_ref[2:3, :]
    inf = jnp.full((1, CBLK), jnp.inf, jnp.float32)
    zi = jnp.zeros((1, CBLK), jnp.int32)
    d1, d2v, d3 = inf, inf, inf
    a1, a2v, a3 = zi, zi, zi
    lane = lax.broadcasted_iota(jnp.int32, (SK, CBLK), 0)
    for ci in range(nchunk):
        sx = sup_ref[0, pl.ds(ci * SK, SK), 0:1]
        sy = sup_ref[0, pl.ds(ci * SK, SK), 1:2]
        sz = sup_ref[0, pl.ds(ci * SK, SK), 2:3]
        dx = cx - sx
        dy = cy - sy
        dz = cz - sz
        dd = dx * dx + dy * dy + dz * dz
        dd = jnp.where(dd <= jnp.float32(RAD2), dd, jnp.float32(BIG))
        idx = lane + ci * SK
        for k in range(3):
            m = jnp.min(dd, axis=0, keepdims=True)
            am = jnp.min(jnp.where(dd == m, idx, N), axis=0, keepdims=True)
            if k < 2:
                dd = jnp.where(idx == am, jnp.float32(TAKEN), dd)
            lt1 = m < d1
            lt2 = m < d2v
            lt3 = m < d3
            nd1 = jnp.where(lt1, m, d1)
            na1 = jnp.where(lt1, am, a1)
            nd2 = jnp.where(lt1, d1, jnp.where(lt2, m, d2v))
            na2 = jnp.where(lt1, a1, jnp.where(lt2, am, a2v))
            nd3 = jnp.where(lt2, d2v, jnp.where(lt3, m, d3))
            na3 = jnp.where(lt2, a2v, jnp.where(lt3, am, a3))
            d1, d2v, d3, a1, a2v, a3 = nd1, nd2, nd3, na1, na2, na3
    s1 = jnp.minimum(jnp.sqrt(jnp.maximum(d1, 0.0)), jnp.float32(1e8))
    s2 = jnp.minimum(jnp.sqrt(jnp.maximum(d2v, 0.0)), jnp.float32(1e8))
    s3 = jnp.minimum(jnp.sqrt(jnp.maximum(d3, 0.0)), jnp.float32(1e8))
    r1 = 1.0 / (s1 + jnp.float32(1e-8))
    r2 = 1.0 / (s2 + jnp.float32(1e-8))
    r3 = 1.0 / (s3 + jnp.float32(1e-8))
    norm = jnp.maximum(r1 + r2 + r3, jnp.float32(1e-8))
    w1 = r1 / norm
    w2 = r2 / norm
    w3 = r3 / norm
    keep = jnp.where(d1 > jnp.float32(1e20), 0.0, 1.0)
    zf = jnp.zeros((1, CBLK), jnp.float32)
    pw_ref[...] = jnp.concatenate([w1, w2, w3, cx, cy, cz, keep, zf], axis=0)
    pi_ref[...] = jnp.concatenate(
        [a1 + s0, a2v + s0, a3 + s0, zi, zi, zi, zi, zi], axis=0)


def _run_nn(cflatT, sup3, interpret=False):
    nblk = CEN // CBLK
    mpb = M // CBLK
    qpb = M // B // CBLK
    return pl.pallas_call(
        _nn_body,
        grid=(nblk,),
        in_specs=[
            pl.BlockSpec((3, CBLK), lambda i: (0, i)),
            pl.BlockSpec((1, NB, 3), lambda i: ((i % mpb) // qpb, 0, 0)),
        ],
        out_specs=[
            pl.BlockSpec((8, CBLK), lambda i: (0, i)),
            pl.BlockSpec((8, CBLK), lambda i: (0, i)),
        ],
        out_shape=[
            jax.ShapeDtypeStruct((8, CEN), jnp.float32),
            jax.ShapeDtypeStruct((8, CEN), jnp.int32),
        ],
        interpret=interpret,
    )(cflatT, sup3)


SC_ROWS = SC_CHUNK // 128


def _sc_assemble(tabs, pws, pis):
    mesh = plsc.VectorSubcoreMesh(core_axis_name="c", subcore_axis_name="s")
    per_w = CEN // NWORK
    nch = per_w // SC_CHUNK

    @functools.partial(
        pl.kernel, mesh=mesh,
        out_type=[jax.ShapeDtypeStruct((CEN,), jnp.float32)
                  for _ in range(11)],
        scratch_types=(
            [pltpu.VMEM((SC_CHUNK,), jnp.float32) for _ in range(7)]
            + [pltpu.VMEM((SC_ROWS, 128), jnp.int32) for _ in range(3)]
            + [pltpu.VMEM((SC_CHUNK,), jnp.float32) for _ in range(15)]
            + [pltpu.VMEM((SC_CHUNK,), jnp.float32) for _ in range(11)]
            + [pltpu.SemaphoreType.DMA]
        ),
    )
    def k(*refs):
        tab_h = refs[0:5]
        pw_h = refs[5:12]
        pi_h = refs[12:15]
        out_h = refs[15:26]
        pw_v = refs[26:33]
        pi_v = refs[33:36]
        g_v = refs[36:51]
        nf_v = refs[51:62]
        sem = refs[62]
        wid = lax.axis_index("s") * 2 + lax.axis_index("c")
        for ch in range(nch):
            base = pl.multiple_of(wid * per_w + ch * SC_CHUNK, 8)
            for f in range(7):
                pltpu.sync_copy(pw_h[f].at[pl.ds(base, SC_CHUNK)], pw_v[f])
            for f in range(3):
                for j in range(SC_ROWS):
                    pltpu.sync_copy(
                        pi_h[f].at[pl.ds(base + j * 128, 128)],
                        pi_v[f].at[j])
            copies = []
            for kk in range(3):
                for f in range(5):
                    for j in range(SC_ROWS):
                        copies.append(pltpu.async_copy(
                            tab_h[f].at[pi_v[kk].at[j]],
                            g_v[kk * 5 + f].at[pl.ds(j * 128, 128)],
                            sem))
            for c in copies:
                c.wait()

            def body(i, carry):
                sl = pl.ds(i * 16, 16)
                w3v = (pw_v[0][sl], pw_v[1][sl], pw_v[2][sl])
                ccx = pw_v[3][sl]
                ccy = pw_v[4][sl]
                ccz = pw_v[5][sl]
                keep = pw_v[6][sl]
                it0 = jnp.zeros((16,), jnp.float32)
                it1 = jnp.zeros((16,), jnp.float32)
                for kk in range(3):
                    gx = g_v[kk * 5 + 0][sl]
                    gy = g_v[kk * 5 + 1][sl]
                    gz = g_v[kk * 5 + 2][sl]
                    gf0 = g_v[kk * 5 + 3][sl]
                    gf1 = g_v[kk * 5 + 4][sl]
                    it0 = it0 + w3v[kk] * gf0
                    it1 = it1 + w3v[kk] * gf1
                    nf_v[2 + 3 * kk][sl] = (ccx - gx) * keep
                    nf_v[3 + 3 * kk][sl] = (ccy - gy) * keep
                    nf_v[4 + 3 * kk][sl] = (ccz - gz) * keep
                nf_v[0][sl] = it0 * keep
                nf_v[1][sl] = it1 * keep
                return carry

            lax.fori_loop(0, SC_CHUNK // 16, body, 0)
            for f in range(11):
                pltpu.sync_copy(nf_v[f], out_h[f].at[pl.ds(base, SC_CHUNK)])

    return k(*tabs, *pws, *pis)


def _mlp_a_body(nf_ref, W1_ref, g1_ref, b1_ref, out_ref):
    RB = min(512, M)
    w1 = W1_ref[0]
    g1v = g1_ref[0]
    b1v = b1_ref[0]
    s = jnp.zeros((1, C_AGG), jnp.float32)
    s2 = jnp.zeros((1, C_AGG), jnp.float32)
    dn = (((0,), (0,)), ((), ()))
    for rb in range(M // RB):
        xT = nf_ref[0, :, pl.ds(rb * RB, RB)]
        h = lax.dot_general(xT, w1, dn, preferred_element_type=jnp.float32)
        s = s + jnp.sum(h, axis=0, keepdims=True)
        s2 = s2 + jnp.sum(h * h, axis=0, keepdims=True)
    mu = s / M
    var = s2 / M - mu * mu
    scale = g1v / jnp.sqrt(var + jnp.float32(1e-5))
    shift = b1v - mu * scale
    for rb in range(M // RB):
        xT = nf_ref[0, :, pl.ds(rb * RB, RB)]
        h = lax.dot_general(xT, w1, dn, preferred_element_type=jnp.float32)
        out_ref[0, pl.ds(rb * RB, RB), :] = jnp.maximum(h * scale + shift,
                                                        0.0)


def _mlp_b1_body(h_ref, W2_ref, acc_ref):
    xs = [h_ref[v] for v in range(TOTAL_VOX)]
    x = jnp.concatenate(xs, axis=1)
    acc_ref[...] = jnp.dot(x, W2_ref[...],
                           preferred_element_type=jnp.float32)


def _mlp_b2_body(acc_ref, g2_ref, b2_ref, out_ref):
    RB = min(512, M)
    s = jnp.zeros((1, POST), jnp.float32)
    s2 = jnp.zeros((1, POST), jnp.float32)
    for rb in range(M // RB):
        part = acc_ref[pl.ds(rb * RB, RB), :]
        s = s + jnp.sum(part, axis=0, keepdims=True)
        s2 = s2 + jnp.sum(part * part, axis=0, keepdims=True)
    mu = s / M
    var = s2 / M - mu * mu
    scale = g2_ref[...] / jnp.sqrt(var + jnp.float32(1e-5))
    shift = b2_ref[...] - mu * scale
    for rb in range(M // RB):
        h = acc_ref[pl.ds(rb * RB, RB), :]
        out_ref[pl.ds(rb * RB, RB), :] = jnp.maximum(h * scale + shift, 0.0)


def _run_mlp(nfT, W1, g1, b1, W2, g2, b2, interpret=False):
    h = pl.pallas_call(
        _mlp_a_body,
        grid=(TOTAL_VOX,),
        in_specs=[
            pl.BlockSpec((1, CR, M), lambda v: (v, 0, 0)),
            pl.BlockSpec((1, CR, C_AGG), lambda v: (v, 0, 0)),
            pl.BlockSpec((1, 1, C_AGG), lambda v: (v, 0, 0)),
            pl.BlockSpec((1, 1, C_AGG), lambda v: (v, 0, 0)),
        ],
        out_specs=pl.BlockSpec((1, M, C_AGG), lambda v: (v, 0, 0)),
        out_shape=jax.ShapeDtypeStruct((TOTAL_VOX, M, C_AGG), jnp.float32),
        interpret=interpret,
    )(nfT, W1, g1.reshape(TOTAL_VOX, 1, C_AGG), b1.reshape(TOTAL_VOX, 1, C_AGG))
    RB = min(512, M)
    acc = pl.pallas_call(
        _mlp_b1_body,
        grid=(M // RB,),
        in_specs=[
            pl.BlockSpec((TOTAL_VOX, RB, C_AGG), lambda r: (0, r, 0)),
            pl.BlockSpec((TOTAL_VOX * C_AGG, POST), lambda r: (0, 0)),
        ],
        out_specs=pl.BlockSpec((RB, POST), lambda r: (r, 0)),
        out_shape=jax.ShapeDtypeStruct((M, POST), jnp.float32),
        interpret=interpret,
    )(h, W2)
    return pl.pallas_call(
        _mlp_b2_body,
        in_specs=[
            pl.BlockSpec((M, POST), lambda: (0, 0)),
            pl.BlockSpec((1, POST), lambda: (0, 0)),
            pl.BlockSpec((1, POST), lambda: (0, 0)),
        ],
        out_specs=pl.BlockSpec((M, POST), lambda: (0, 0)),
        out_shape=jax.ShapeDtypeStruct((M, POST), jnp.float32),
        interpret=interpret,
    )(acc, g2.reshape(1, POST), b2.reshape(1, POST))


def _impl(support_xyz, support_features, new_xyz, W1, g1, b1, W2, g2, b2,
          interpret=False):
    centers = _voxel_offs()[:, None, :] + new_xyz[None, :, :]
    cflatT = centers.reshape(-1, 3).T
    sup3 = support_xyz.reshape(B, NB, 3)
    tab = _run_table(support_xyz.T, support_features.T, interpret=interpret)
    pw, pi = _run_nn(cflatT, sup3, interpret=interpret)
    nfs = _sc_assemble([tab[f] for f in range(5)],
                       [pw[f] for f in range(7)],
                       [pi[f] for f in range(3)])
    nf3 = jnp.stack([x.reshape(TOTAL_VOX, M) for x in nfs], axis=1)
    return _run_mlp(nf3, W1, g1, b1, W2, g2, b2, interpret=interpret)


def kernel(support_xyz, support_features, batch_num_xyzs, new_xyz,
           batch_num_new_xyzs, W1, g1, b1, W2, g2, b2):
    return _impl(support_xyz, support_features, new_xyz, W1, g1, b1, W2, g2,
                 b2)
